# baseline (speedup 1.0000x reference)
import functools

import jax
import jax.numpy as jnp
from jax import lax
from jax.experimental import pallas as pl
from jax.experimental.pallas import tpu as pltpu
from jax.experimental.pallas import tpu_sc as plsc

N = 10000
C = 128
OUT = 128
K = 32

NW = 32
NPAD = 10240
PER_W = NPAD // NW
CHUNK = 4
NCHUNK = PER_W // CHUNK
LG = OUT // 16


def _mm_body(x_ref, wc_ref, b_ref, a_ref, t_ref):
    i = pl.program_id(0)
    xb = x_ref[...]
    rid = lax.broadcasted_iota(jnp.int32, (_MM_BLK, 1), 0) + i * _MM_BLK
    xb = jnp.where(rid < N, xb, 1e6)
    dn = (((1,), (1,)), ((), ()))
    h = lax.dot_general(xb, wc_ref[...], dn, preferred_element_type=jnp.float32)
    a_ref[...] = h[:, :OUT] + b_ref[...]
    t_ref[...] = h[:, OUT:]


_MM_BLK = 2048


def _mm(x, wc, b2):
    return pl.pallas_call(
        _mm_body,
        grid=(NPAD // _MM_BLK,),
        in_specs=[
            pl.BlockSpec((_MM_BLK, C), lambda i: (i, 0)),
            pl.BlockSpec((2 * OUT, C), lambda i: (0, 0)),
            pl.BlockSpec((1, OUT), lambda i: (0, 0)),
        ],
        out_specs=[
            pl.BlockSpec((_MM_BLK, OUT), lambda i: (i, 0)),
            pl.BlockSpec((_MM_BLK, OUT), lambda i: (i, 0)),
        ],
        out_shape=[
            jax.ShapeDtypeStruct((NPAD, OUT), jnp.float32),
            jax.ShapeDtypeStruct((NPAD, OUT), jnp.float32),
        ],
    )(x, wc, b2)


@functools.partial(
    pl.kernel,
    out_type=jax.ShapeDtypeStruct((N, OUT), jnp.float32),
    mesh=plsc.VectorSubcoreMesh(core_axis_name="c", subcore_axis_name="s"),
    compiler_params=pltpu.CompilerParams(needs_layout_passes=False),
    scratch_types=[
        pltpu.VMEM_SHARED((NPAD, OUT), jnp.float32),
        pltpu.VMEM((PER_W * K,), jnp.int32),
        pltpu.VMEM((CHUNK * K, OUT), jnp.float32),
        pltpu.VMEM((CHUNK * K, OUT), jnp.float32),
        pltpu.VMEM((2 * CHUNK, OUT), jnp.float32),
        pltpu.VMEM((2 * CHUNK, OUT), jnp.float32),
        pltpu.VMEM((2 * CHUNK, OUT), jnp.float32),
        pltpu.VMEM((2 * CHUNK, OUT), jnp.float32),
        pltpu.SemaphoreType.DMA,
        pltpu.SemaphoreType.DMA,
        pltpu.SemaphoreType.DMA,
        pltpu.SemaphoreType.DMA,
        pltpu.SemaphoreType.DMA,
        pltpu.SemaphoreType.DMA,
    ],
)
def _sc_gather_max(
    t_hbm, idx_hbm, a_hbm, out_hbm,
    t_sh, idx_v, gbuf0, gbuf1, abuf0, abuf1, obuf0, obuf1,
    gsem0, gsem1, asem0, asem1, osem0, osem1,
):
    sid = lax.axis_index("s")
    wid = sid * 2 + lax.axis_index("c")
    base = jnp.minimum(wid * PER_W, N - PER_W)
    rows = NPAD // 16
    pltpu.sync_copy(
        t_hbm.at[pl.ds(sid * rows, rows)], t_sh.at[pl.ds(sid * rows, rows)]
    )
    pltpu.sync_copy(idx_hbm.at[pl.ds(base * K, PER_W * K)], idx_v)
    plsc.subcore_barrier()

    gbufs = (gbuf0, gbuf1)
    abufs = (abuf0, abuf1)
    obufs = (obuf0, obuf1)
    gsems = (gsem0, gsem1)
    asems = (asem0, asem1)
    osems = (osem0, osem1)

    def gstart(ch, slot):
        pltpu.async_copy(
            t_sh.at[idx_v.at[pl.ds(ch * (CHUNK * K), CHUNK * K)]],
            gbufs[slot], gsems[slot],
        )

    def gstart_if(ch, slot):
        @pl.when(ch < NCHUNK)
        def _():
            gstart(ch, slot)

    def astart(pair, slot):
        pltpu.async_copy(
            a_hbm.at[pl.ds(base + pair * (2 * CHUNK), 2 * CHUNK)],
            abufs[slot], asems[slot],
        )

    def finish(ch, slot, pslot, prow):
        gbuf, abuf, obuf = gbufs[slot], abufs[pslot], obufs[pslot]
        pltpu.make_async_copy(
            t_sh.at[idx_v.at[pl.ds(ch * (CHUNK * K), CHUNK * K)]],
            gbuf, gsems[slot],
        ).wait()

        def node_body(n, carry2):
            rb = n * K
            for g in range(LG):
                sl = pl.ds(g * 16, 16)
                acc = gbuf[rb, sl]
                for k in range(1, K):
                    acc = jnp.maximum(acc, gbuf[rb + k, sl])
                obuf[prow + n, sl] = jnp.maximum(acc + abuf[prow + n, sl], 0.0)
            return carry2

        lax.fori_loop(0, CHUNK, node_body, 0)

    def await_a(pslot):
        pltpu.make_async_copy(
            a_hbm.at[pl.ds(base, 2 * CHUNK)], abufs[pslot], asems[pslot]
        ).wait()

    def await_store(pslot):
        pltpu.make_async_copy(
            obufs[pslot], out_hbm.at[pl.ds(base, 2 * CHUNK)], osems[pslot]
        ).wait()

    def store(pair, pslot):
        pltpu.async_copy(
            obufs[pslot],
            out_hbm.at[pl.ds(base + pair * (2 * CHUNK), 2 * CHUNK)],
            osems[pslot],
        )

    gstart(0, 0)
    gstart(1, 1)
    astart(0, 0)

    def quad_body(q, carry):
        c0 = 4 * q
        astart(2 * q + 1, 1)
        await_a(0)

        @pl.when(q > 0)
        def _():
            await_store(0)

        finish(c0, 0, 0, 0)
        gstart_if(c0 + 2, 0)
        finish(c0 + 1, 1, 0, CHUNK)
        gstart_if(c0 + 3, 1)
        store(2 * q, 0)

        @pl.when(2 * q + 2 < NCHUNK // 2)
        def _():
            astart(2 * q + 2, 0)

        await_a(1)

        @pl.when(q > 0)
        def _():
            await_store(1)

        finish(c0 + 2, 0, 1, 0)
        gstart_if(c0 + 4, 0)
        finish(c0 + 3, 1, 1, CHUNK)
        gstart_if(c0 + 5, 1)
        store(2 * q + 1, 1)
        return carry

    lax.fori_loop(0, NCHUNK // 4, quad_body, 0)
    await_store(0)
    await_store(1)


def kernel(x, edge_index, W, b):
    x = x.astype(jnp.float32)
    W = W.astype(jnp.float32)
    idx = edge_index[0].astype(jnp.int32).reshape(N * K)
    w1 = W[:, :C]
    w2 = W[:, C:]
    wc = jnp.concatenate([w1 - w2, w2], axis=0)
    a, t = _mm(x, wc, b.astype(jnp.float32).reshape(1, OUT))
    return _sc_gather_max(t, idx, a)


# speedup vs baseline: 1.1207x; 1.1207x over previous
import functools

import jax
import jax.numpy as jnp
from jax import lax
from jax.experimental import pallas as pl
from jax.experimental.pallas import tpu as pltpu
from jax.experimental.pallas import tpu_sc as plsc

N = 10000
C = 128
OUT = 128
K = 32

NW = 32
NPAD = 10240
PER_W = NPAD // NW
CHUNK = 4
NCHUNK = PER_W // CHUNK
LG = OUT // 16


def _mm_body(x_ref, wc_ref, b_ref, a_ref, t_ref):
    i = pl.program_id(0)
    xb = x_ref[...]
    rid = lax.broadcasted_iota(jnp.int32, (_MM_BLK, 1), 0) + i * _MM_BLK
    xb = jnp.where(rid < N, xb, 1e6)
    dn = (((1,), (1,)), ((), ()))
    h = lax.dot_general(xb, wc_ref[...], dn, preferred_element_type=jnp.float32)
    a_ref[...] = h[:, :OUT] + b_ref[...]
    t_ref[...] = h[:, OUT:]


_MM_BLK = 2048


def _mm(x, wc, b2):
    return pl.pallas_call(
        _mm_body,
        grid=(NPAD // _MM_BLK,),
        in_specs=[
            pl.BlockSpec((_MM_BLK, C), lambda i: (i, 0)),
            pl.BlockSpec((2 * OUT, C), lambda i: (0, 0)),
            pl.BlockSpec((1, OUT), lambda i: (0, 0)),
        ],
        out_specs=[
            pl.BlockSpec((_MM_BLK, OUT), lambda i: (i, 0)),
            pl.BlockSpec((_MM_BLK, OUT), lambda i: (i, 0)),
        ],
        out_shape=[
            jax.ShapeDtypeStruct((NPAD, OUT), jnp.float32),
            jax.ShapeDtypeStruct((NPAD, OUT), jnp.float32),
        ],
    )(x, wc, b2)


@functools.partial(
    pl.kernel,
    out_type=jax.ShapeDtypeStruct((N, OUT), jnp.float32),
    mesh=plsc.VectorSubcoreMesh(core_axis_name="c", subcore_axis_name="s"),
    compiler_params=pltpu.CompilerParams(needs_layout_passes=False),
    scratch_types=[
        pltpu.VMEM_SHARED((NPAD, OUT), jnp.float32),
        pltpu.VMEM((PER_W * K,), jnp.int32),
        pltpu.VMEM((CHUNK * K, OUT), jnp.float32),
        pltpu.VMEM((CHUNK * K, OUT), jnp.float32),
        pltpu.VMEM((CHUNK, OUT), jnp.float32),
        pltpu.VMEM((CHUNK, OUT), jnp.float32),
        pltpu.VMEM((CHUNK, OUT), jnp.float32),
        pltpu.VMEM((CHUNK, OUT), jnp.float32),
        pltpu.SemaphoreType.DMA,
        pltpu.SemaphoreType.DMA,
        pltpu.SemaphoreType.DMA,
        pltpu.SemaphoreType.DMA,
        pltpu.SemaphoreType.DMA,
        pltpu.SemaphoreType.DMA,
    ],
)
def _sc_gather_max(
    t_hbm, idx_hbm, a_hbm, out_hbm,
    t_sh, idx_v, gbuf0, gbuf1, abuf0, abuf1, obuf0, obuf1,
    gsem0, gsem1, asem0, asem1, osem0, osem1,
):
    sid = lax.axis_index("s")
    wid = sid * 2 + lax.axis_index("c")
    base = jnp.minimum(wid * PER_W, N - PER_W)
    rows = NPAD // 16
    pltpu.sync_copy(
        t_hbm.at[pl.ds(sid * rows, rows)], t_sh.at[pl.ds(sid * rows, rows)]
    )
    pltpu.sync_copy(idx_hbm.at[pl.ds(base * K, PER_W * K)], idx_v)
    plsc.subcore_barrier()

    gbufs = (gbuf0, gbuf1)
    abufs = (abuf0, abuf1)
    obufs = (obuf0, obuf1)
    gsems = (gsem0, gsem1)
    asems = (asem0, asem1)
    osems = (osem0, osem1)

    def start(ch, slot):
        pltpu.async_copy(
            t_sh.at[idx_v.at[pl.ds(ch * (CHUNK * K), CHUNK * K)]],
            gbufs[slot], gsems[slot],
        )
        pltpu.async_copy(
            a_hbm.at[pl.ds(base + ch * CHUNK, CHUNK)], abufs[slot], asems[slot]
        )

    def finish(ch, slot, wait_store):
        gbuf, abuf, obuf = gbufs[slot], abufs[slot], obufs[slot]
        pltpu.make_async_copy(
            t_sh.at[idx_v.at[pl.ds(ch * (CHUNK * K), CHUNK * K)]],
            gbuf, gsems[slot],
        ).wait()
        pltpu.make_async_copy(
            a_hbm.at[pl.ds(base + ch * CHUNK, CHUNK)], abuf, asems[slot]
        ).wait()

        @pl.when(wait_store)
        def _():
            pltpu.make_async_copy(
                obuf, out_hbm.at[pl.ds(base, CHUNK)], osems[slot]
            ).wait()

        def node_body(n, carry2):
            rb = n * K
            for g in range(LG):
                sl = pl.ds(g * 16, 16)
                acc = gbuf[rb, sl]
                for k in range(1, K):
                    acc = jnp.maximum(acc, gbuf[rb + k, sl])
                obuf[n, sl] = jnp.maximum(acc + abuf[n, sl], 0.0)
            return carry2

        lax.fori_loop(0, CHUNK, node_body, 0)
        pltpu.async_copy(
            obuf, out_hbm.at[pl.ds(base + ch * CHUNK, CHUNK)], osems[slot]
        )

    start(0, 0)
    start(1, 1)

    def pair_body(p, carry):
        ch0 = 2 * p
        finish(ch0, 0, p > 0)

        @pl.when(ch0 + 2 < NCHUNK)
        def _():
            start(ch0 + 2, 0)

        finish(ch0 + 1, 1, p > 0)

        @pl.when(ch0 + 3 < NCHUNK)
        def _():
            start(ch0 + 3, 1)

        return carry

    lax.fori_loop(0, NCHUNK // 2, pair_body, 0)
    pltpu.make_async_copy(obuf0, out_hbm.at[pl.ds(base, CHUNK)], osem0).wait()
    pltpu.make_async_copy(obuf1, out_hbm.at[pl.ds(base, CHUNK)], osem1).wait()


def kernel(x, edge_index, W, b):
    x = x.astype(jnp.float32)
    W = W.astype(jnp.float32)
    idx = edge_index[0].astype(jnp.int32).reshape(N * K)
    w1 = W[:, :C]
    w2 = W[:, C:]
    wc = jnp.concatenate([w1 - w2, w2], axis=0)
    a, t = _mm(x, wc, b.astype(jnp.float32).reshape(1, OUT))
    return _sc_gather_max(t, idx, a)
